# Initial kernel scaffold; baseline (speedup 1.0000x reference)
import jax
import jax.numpy as jnp
from jax.experimental import pallas as pl


def kernel(idxs, frame_embs):
    raise NotImplementedError("write your pallas kernel here")


# speedup vs baseline: 2.2872x; 2.2872x over previous
import functools

import jax
import jax.numpy as jnp
import numpy as np
from jax.experimental import pallas as pl
from jax.experimental.pallas import tpu as pltpu

N_FRAMES = 240
HEIGHT = 32
WIDTH = 32
N_DIMS = 64
KSIZE = 5
PAD = KSIZE // 2
TEMP = 5.0
MAX_NORM = float(N_DIMS)
ROW = HEIGHT * WIDTH * N_DIMS
SUB = 8
LANE = ROW // SUB

_W = np.exp(-((np.arange(KSIZE, dtype=np.float32) - PAD) ** 2) / np.float32(TEMP))
_W = (_W / _W.sum()).astype(np.float32)


def _body(idx_ref, t0, t1, t2, t3, t4, out_ref):
    acc = jnp.zeros((SUB, LANE), jnp.float32)
    for k, tap in enumerate((t0, t1, t2, t3, t4)):
        x = tap[0]
        norm = jnp.sqrt(jnp.sum(x * x))
        scale = jnp.minimum(jnp.float32(1.0), MAX_NORM / (norm + 1e-7))
        acc = acc + (_W[k] * scale) * x
    out_ref[0] = acc


def kernel(idxs, frame_embs):
    B = idxs.shape[0]
    table = frame_embs.reshape(-1, SUB, LANE)

    def tap_spec(k):
        return pl.BlockSpec(
            (1, SUB, LANE), lambda b, idx_ref, k=k: (idx_ref[b] + k, 0, 0)
        )

    grid_spec = pltpu.PrefetchScalarGridSpec(
        num_scalar_prefetch=1,
        grid=(B,),
        in_specs=[tap_spec(k) for k in range(KSIZE)],
        out_specs=pl.BlockSpec((1, SUB, LANE), lambda b, idx_ref: (b, 0, 0)),
    )
    out = pl.pallas_call(
        _body,
        grid_spec=grid_spec,
        out_shape=jax.ShapeDtypeStruct((B, SUB, LANE), jnp.float32),
    )(idxs.astype(jnp.int32), table, table, table, table, table)
    return out.reshape(B, N_DIMS, HEIGHT, WIDTH)
